# baseline (speedup 1.0000x reference)
import functools

import jax
import jax.numpy as jnp
from jax import lax
from jax.experimental import pallas as pl
from jax.experimental.pallas import tpu as pltpu
from jax.experimental.pallas import tpu_sc as plsc

_NC = 2
_NS = 16
_NW = _NC * _NS
_LANES = 16
_T = 8
_GRP = 4


def _when(cond):
    if isinstance(cond, bool):
        return (lambda f: f() if cond else None)
    return pl.when(cond)


def _compose_body(nt, d, tpw, nsamp, idx_hbm, idx2d_hbm, len_hbm, table_hbm,
                  out_hbm, mask_hbm, idx_all, idx_mine, qidx, mask_v, len_v,
                  gbufs, rows, obufs, zbuf, qsem, gsems, osems, zsem):
    nslots = tpw // _T
    lsz = nt // nsamp
    cid = lax.axis_index("c")
    sid = lax.axis_index("s")
    wid = sid * _NC + cid
    g0 = wid * tpw

    pltpu.sync_copy(len_hbm, len_v)
    pltpu.sync_copy(idx_hbm.at[pl.ds(g0 * 4, tpw * 4)], idx_all)
    lens_vec = len_v[...]
    lane = lax.iota(jnp.int32, _LANES)
    zeros = jnp.zeros((_LANES,), jnp.float32)

    for h in range(nslots // 2 // _LANES):
        qidx[pl.ds(h * _LANES, _LANES)] = (wid + _NW * (lane + h * _LANES)) >> 1
    pltpu.async_copy(idx2d_hbm.at[qidx], idx_mine, qsem)
    col0 = (wid % 2) * (8 * _T)

    for t in range(2 * _T):
        def zinit(dd, c, t=t):
            sl = pl.ds(pl.multiple_of(dd * _LANES, _LANES), _LANES)
            zbuf[t, sl] = zeros
            return c
        lax.fori_loop(0, d // _LANES, zinit, 0, unroll=4)

    lbs = [jnp.max(jnp.where(lane == s, lens_vec, 0)) for s in range(nsamp)]

    def lb_of(sq):
        r = lbs[0]
        for s in range(1, nsamp):
            r = jnp.where(sq == s, lbs[s], r)
        return r

    def slot_nv(j):
        gq = (wid + _NW * (j >> 1)) * (2 * _T) + (j & 1) * _T
        sq = gq // lsz
        return jnp.minimum(jnp.maximum(lb_of(sq) - (gq - sq * lsz), 0), _T)

    def pair_nv(p):
        gp = (wid + _NW * p) * (2 * _T)
        sq = gp // lsz
        return jnp.minimum(jnp.maximum(lb_of(sq) - (gp - sq * lsz), 0), 2 * _T)

    wpersamp = _NW // nsamp
    b = wid // wpersamp
    r0 = (wid % wpersamp) * tpw
    nv = jnp.minimum(jnp.maximum(lb_of(b) - r0, 0), tpw)

    def mask_grp(grp, carry):
        t = lane + grp * _LANES
        p = t * 4
        v = plsc.load_gather(idx_all, [p])
        for j in range(1, 4):
            v = v | plsc.load_gather(idx_all, [p + j])
        m = ((v != 0) & (t < nv)).astype(jnp.int32)
        mask_v[pl.ds(pl.multiple_of(grp * _LANES, _LANES), _LANES)] = m
        return carry

    lax.fori_loop(0, tpw // _LANES, mask_grp, 0)
    pltpu.sync_copy(mask_v, mask_hbm.at[pl.ds(g0, tpw)])

    pltpu.make_async_copy(idx2d_hbm.at[qidx], idx_mine, qsem).wait()

    def prep_gather(j, kr):
        nvq = slot_nv(j)
        rvec = jnp.full((_LANES,), j >> 1, jnp.int32)
        cb = col0 + (j & 1) * (4 * _T)
        for h in range(4 * _T // _LANES):
            tok = (lane >> 2) + 4 * h
            v = plsc.load_gather(idx_mine, [rvec, cb + lane + h * _LANES])
            gbufs[kr][pl.ds(h * _LANES, _LANES)] = jnp.where(tok < nvq, v, 0)

        @pl.when(nvq > 0)
        def _():
            pltpu.async_copy(table_hbm.at[gbufs[kr]], rows[kr], gsems[kr])

    def wait_gather(kr, nvq):
        @pl.when(nvq > 0)
        def _():
            pltpu.make_async_copy(table_hbm.at[gbufs[kr]], rows[kr],
                                  gsems[kr]).wait()

    def wait_write(p, ko):
        nvp = pair_nv(p)

        @pl.when(nvp > 0)
        def _():
            pltpu.make_async_copy(obufs[ko], out_hbm.at[pl.ds(0, 2 * _T), :],
                                  osems[ko]).wait()

        @pl.when(nvp == 0)
        def _():
            pltpu.make_async_copy(zbuf, out_hbm.at[pl.ds(0, 2 * _T), :],
                                  zsem).wait()

    def slot_step(s, k):
        kr = k % 2
        h = k & 1
        ko = (k >> 1) & 1
        nvq = slot_nv(s)
        p = s >> 1

        _when(s + 1 < nslots)(lambda: prep_gather(s + 1, 1 - kr))
        wait_gather(kr, nvq)
        if h == 0:
            _when(s >= 4)(lambda: wait_write(p - 2, ko))

        @pl.when(nvq > 0)
        def _():
            for t in range(_T):
                @plsc.parallel_loop(0, d // _LANES, unroll=16)
                def _(dd, t=t):
                    sl = pl.ds(pl.multiple_of(dd * _LANES, _LANES), _LANES)
                    rws = rows[kr]
                    obufs[ko][h * _T + t, sl] = (
                        (rws[4 * t, sl] + rws[4 * t + 1, sl]) +
                        (rws[4 * t + 2, sl] + rws[4 * t + 3, sl]))

        if h == 1:
            nvp = pair_nv(p)

            @pl.when((nvq == 0) & (nvp > 0))
            def _():
                for t in range(_T):
                    @plsc.parallel_loop(0, d // _LANES, unroll=16)
                    def _(dd, t=t):
                        sl = pl.ds(pl.multiple_of(dd * _LANES, _LANES),
                                   _LANES)
                        obufs[ko][_T + t, sl] = zeros

            gp = (wid + _NW * p) * (2 * _T)

            @pl.when(nvp > 0)
            def _():
                pltpu.async_copy(obufs[ko], out_hbm.at[pl.ds(gp, 2 * _T), :],
                                 osems[ko])

            @pl.when(nvp == 0)
            def _():
                pltpu.async_copy(zbuf, out_hbm.at[pl.ds(gp, 2 * _T), :], zsem)

    prep_gather(0, 0)
    ngroups = nslots // _GRP

    def group_body(i, carry):
        for k in range(_GRP):
            slot_step(_GRP * i + k, k)
        return carry

    lax.fori_loop(0, ngroups, group_body, 0)
    npairs = nslots // 2
    for p in (npairs - 2, npairs - 1):
        wait_write(p, p % 2)


@functools.partial(jax.jit, static_argnames=("nt", "d", "nsamp"))
def _compose_sc(idx_flat, idx2d, len_pad, table, *, nt, d, nsamp):
    tpw = nt // _NW
    nslots = tpw // _T
    mesh = plsc.VectorSubcoreMesh(
        core_axis_name="c", subcore_axis_name="s",
        num_cores=_NC, num_subcores=_NS)
    body = functools.partial(_compose_body, nt, d, tpw, nsamp)
    return pl.kernel(
        body,
        out_type=[
            jax.ShapeDtypeStruct((nt, d), jnp.float32),
            jax.ShapeDtypeStruct((nt,), jnp.int32),
        ],
        mesh=mesh,
        compiler_params=pltpu.CompilerParams(needs_layout_passes=False),
        scratch_types=[
            pltpu.VMEM((tpw * 4,), jnp.int32),
            pltpu.VMEM((nslots // 2, 16 * _T), jnp.int32),
            pltpu.VMEM((nslots // 2,), jnp.int32),
            pltpu.VMEM((tpw,), jnp.int32),
            pltpu.VMEM((_LANES,), jnp.int32),
            [pltpu.VMEM((4 * _T,), jnp.int32)] * 2,
            [pltpu.VMEM((4 * _T, d), jnp.float32)] * 2,
            [pltpu.VMEM((2 * _T, d), jnp.float32)] * 2,
            pltpu.VMEM((2 * _T, d), jnp.float32),
            pltpu.SemaphoreType.DMA,
            [pltpu.SemaphoreType.DMA] * 2,
            [pltpu.SemaphoreType.DMA] * 2,
            pltpu.SemaphoreType.DMA,
        ],
    )(idx_flat, idx2d, len_pad, table)


def kernel(phoneme_flat, phoneme_token_len, table):
    bsz, pt = phoneme_flat.shape
    lx = pt // 4
    nt = bsz * lx
    d = table.shape[1]
    idx_flat = phoneme_flat.reshape(-1).astype(jnp.int32)
    idx2d = idx_flat.reshape(nt // (4 * _T), 16 * _T)
    len_pad = jnp.zeros((_LANES,), jnp.int32).at[:bsz].set(
        phoneme_token_len.astype(jnp.int32))
    out_flat, mask_i = _compose_sc(idx_flat, idx2d, len_pad, table,
                                   nt=nt, d=d, nsamp=bsz)
    out = out_flat.reshape(bsz, lx, d)
    pf_mask = mask_i.reshape(bsz, lx).astype(bool)
    return out, pf_mask


# speedup vs baseline: 1.1874x; 1.1874x over previous
import functools

import jax
import jax.numpy as jnp
from jax import lax
from jax.experimental import pallas as pl
from jax.experimental.pallas import tpu as pltpu
from jax.experimental.pallas import tpu_sc as plsc

_NC = 2
_NS = 16
_NW = _NC * _NS
_LANES = 16
_T = 8
_GRP = 4


def _when(cond):
    if isinstance(cond, bool):
        return (lambda f: f() if cond else None)
    return pl.when(cond)


def _compose_body(nt, d, tpw, nsamp, idx_hbm, idx2d_hbm, len_hbm, table_hbm,
                  out_hbm, mask_hbm, idx_all, idx_mine, qidx, mask_v, len_v,
                  gbufs, rows, obufs, zbuf, qsem, gsems, osems, zsem):
    nslots = tpw // _T
    lsz = nt // nsamp
    cid = lax.axis_index("c")
    sid = lax.axis_index("s")
    wid = sid * _NC + cid
    g0 = wid * tpw

    pltpu.sync_copy(len_hbm, len_v)
    pltpu.sync_copy(idx_hbm.at[pl.ds(g0 * 4, tpw * 4)], idx_all)
    lens_vec = len_v[...]
    lane = lax.iota(jnp.int32, _LANES)
    zeros = jnp.zeros((_LANES,), jnp.float32)

    for h in range(nslots // 2 // _LANES):
        qidx[pl.ds(h * _LANES, _LANES)] = (wid + _NW * (lane + h * _LANES)) >> 1
    pltpu.async_copy(idx2d_hbm.at[qidx], idx_mine, qsem)
    col0 = (wid % 2) * (8 * _T)

    for t in range(2 * _T):
        def zinit(dd, c, t=t):
            sl = pl.ds(pl.multiple_of(dd * _LANES, _LANES), _LANES)
            zbuf[t, sl] = zeros
            return c
        lax.fori_loop(0, d // _LANES, zinit, 0, unroll=4)

    lbs = [jnp.max(jnp.where(lane == s, lens_vec, 0)) for s in range(nsamp)]

    def lb_of(sq):
        r = lbs[0]
        for s in range(1, nsamp):
            r = jnp.where(sq == s, lbs[s], r)
        return r

    def slot_nv(j):
        gq = (wid + _NW * (j >> 1)) * (2 * _T) + (j & 1) * _T
        sq = gq // lsz
        return jnp.minimum(jnp.maximum(lb_of(sq) - (gq - sq * lsz), 0), _T)

    def pair_nv(p):
        gp = (wid + _NW * p) * (2 * _T)
        sq = gp // lsz
        return jnp.minimum(jnp.maximum(lb_of(sq) - (gp - sq * lsz), 0), 2 * _T)

    wpersamp = _NW // nsamp
    b = wid // wpersamp
    r0 = (wid % wpersamp) * tpw
    nv = jnp.minimum(jnp.maximum(lb_of(b) - r0, 0), tpw)

    def mask_grp(grp, carry):
        t = lane + grp * _LANES
        p = t * 4
        v = plsc.load_gather(idx_all, [p])
        for j in range(1, 4):
            v = v | plsc.load_gather(idx_all, [p + j])
        m = ((v != 0) & (t < nv)).astype(jnp.int32)
        mask_v[pl.ds(pl.multiple_of(grp * _LANES, _LANES), _LANES)] = m
        return carry

    lax.fori_loop(0, tpw // _LANES, mask_grp, 0)
    pltpu.sync_copy(mask_v, mask_hbm.at[pl.ds(g0, tpw)])

    pltpu.make_async_copy(idx2d_hbm.at[qidx], idx_mine, qsem).wait()

    def prep_gather(j, kr):
        nvq = slot_nv(j)
        rvec = jnp.full((_LANES,), j >> 1, jnp.int32)
        cb = col0 + (j & 1) * (4 * _T)
        for h in range(4 * _T // _LANES):
            tok = (lane >> 2) + 4 * h
            v = plsc.load_gather(idx_mine, [rvec, cb + lane + h * _LANES])
            gbufs[kr][pl.ds(h * _LANES, _LANES)] = jnp.where(tok < nvq, v, 0)

        @pl.when(nvq > 0)
        def _():
            pltpu.async_copy(table_hbm.at[gbufs[kr]], rows[kr], gsems[kr])

    def wait_gather(kr, nvq):
        @pl.when(nvq > 0)
        def _():
            pltpu.make_async_copy(table_hbm.at[gbufs[kr]], rows[kr],
                                  gsems[kr]).wait()

    def wait_write(p, ko):
        nvp = pair_nv(p)

        @pl.when(nvp > 0)
        def _():
            pltpu.make_async_copy(obufs[ko], out_hbm.at[pl.ds(0, 2 * _T), :],
                                  osems[ko]).wait()

        @pl.when(nvp == 0)
        def _():
            pltpu.make_async_copy(zbuf, out_hbm.at[pl.ds(0, 2 * _T), :],
                                  zsem).wait()

    def slot_step(s, k):
        kr = k % 2
        h = k & 1
        ko = (k >> 1) & 1
        nvq = slot_nv(s)
        p = s >> 1

        _when(s + 1 < nslots)(lambda: prep_gather(s + 1, 1 - kr))
        wait_gather(kr, nvq)
        if h == 0:
            _when(s >= 4)(lambda: wait_write(p - 2, ko))

        @pl.when(nvq > 0)
        def _():
            for t in range(_T):
                @plsc.parallel_loop(0, d // _LANES, unroll=4)
                def _(dd, t=t):
                    sl = pl.ds(pl.multiple_of(dd * _LANES, _LANES), _LANES)
                    rws = rows[kr]
                    obufs[ko][h * _T + t, sl] = (
                        (rws[4 * t, sl] + rws[4 * t + 1, sl]) +
                        (rws[4 * t + 2, sl] + rws[4 * t + 3, sl]))

        if h == 1:
            nvp = pair_nv(p)

            @pl.when((nvq == 0) & (nvp > 0))
            def _():
                for t in range(_T):
                    @plsc.parallel_loop(0, d // _LANES, unroll=4)
                    def _(dd, t=t):
                        sl = pl.ds(pl.multiple_of(dd * _LANES, _LANES),
                                   _LANES)
                        obufs[ko][_T + t, sl] = zeros

            gp = (wid + _NW * p) * (2 * _T)

            @pl.when(nvp > 0)
            def _():
                pltpu.async_copy(obufs[ko], out_hbm.at[pl.ds(gp, 2 * _T), :],
                                 osems[ko])

            @pl.when(nvp == 0)
            def _():
                pltpu.async_copy(zbuf, out_hbm.at[pl.ds(gp, 2 * _T), :], zsem)

    prep_gather(0, 0)
    ngroups = nslots // _GRP

    def group_body(i, carry):
        for k in range(_GRP):
            slot_step(_GRP * i + k, k)
        return carry

    lax.fori_loop(0, ngroups, group_body, 0)
    npairs = nslots // 2
    for p in (npairs - 2, npairs - 1):
        wait_write(p, p % 2)


@functools.partial(jax.jit, static_argnames=("nt", "d", "nsamp"))
def _compose_sc(idx_flat, idx2d, len_pad, table, *, nt, d, nsamp):
    tpw = nt // _NW
    nslots = tpw // _T
    mesh = plsc.VectorSubcoreMesh(
        core_axis_name="c", subcore_axis_name="s",
        num_cores=_NC, num_subcores=_NS)
    body = functools.partial(_compose_body, nt, d, tpw, nsamp)
    return pl.kernel(
        body,
        out_type=[
            jax.ShapeDtypeStruct((nt, d), jnp.float32),
            jax.ShapeDtypeStruct((nt,), jnp.int32),
        ],
        mesh=mesh,
        compiler_params=pltpu.CompilerParams(needs_layout_passes=False),
        scratch_types=[
            pltpu.VMEM((tpw * 4,), jnp.int32),
            pltpu.VMEM((nslots // 2, 16 * _T), jnp.int32),
            pltpu.VMEM((nslots // 2,), jnp.int32),
            pltpu.VMEM((tpw,), jnp.int32),
            pltpu.VMEM((_LANES,), jnp.int32),
            [pltpu.VMEM((4 * _T,), jnp.int32)] * 2,
            [pltpu.VMEM((4 * _T, d), jnp.float32)] * 2,
            [pltpu.VMEM((2 * _T, d), jnp.float32)] * 2,
            pltpu.VMEM((2 * _T, d), jnp.float32),
            pltpu.SemaphoreType.DMA,
            [pltpu.SemaphoreType.DMA] * 2,
            [pltpu.SemaphoreType.DMA] * 2,
            pltpu.SemaphoreType.DMA,
        ],
    )(idx_flat, idx2d, len_pad, table)


def kernel(phoneme_flat, phoneme_token_len, table):
    bsz, pt = phoneme_flat.shape
    lx = pt // 4
    nt = bsz * lx
    d = table.shape[1]
    idx_flat = phoneme_flat.reshape(-1).astype(jnp.int32)
    idx2d = idx_flat.reshape(nt // (4 * _T), 16 * _T)
    len_pad = jnp.zeros((_LANES,), jnp.int32).at[:bsz].set(
        phoneme_token_len.astype(jnp.int32))
    out_flat, mask_i = _compose_sc(idx_flat, idx2d, len_pad, table,
                                   nt=nt, d=d, nsamp=bsz)
    out = out_flat.reshape(bsz, lx, d)
    pf_mask = mask_i.reshape(bsz, lx).astype(bool)
    return out, pf_mask


# speedup vs baseline: 1.1975x; 1.0084x over previous
import functools

import jax
import jax.numpy as jnp
from jax import lax
from jax.experimental import pallas as pl
from jax.experimental.pallas import tpu as pltpu
from jax.experimental.pallas import tpu_sc as plsc

_NC = 2
_NS = 16
_NW = _NC * _NS
_LANES = 16
_T = 8
_GRP = 4


def _when(cond):
    if isinstance(cond, bool):
        return (lambda f: f() if cond else None)
    return pl.when(cond)


def _compose_body(nt, d, tpw, nsamp, idx_hbm, idx2d_hbm, len_hbm, table_hbm,
                  out_hbm, mask_hbm, idx_all, idx_mine, qidx, mask_v, len_v,
                  gbufs, rows, obufs, zbuf, qsem, gsems, osems, zsem):
    nslots = tpw // _T
    lsz = nt // nsamp
    cid = lax.axis_index("c")
    sid = lax.axis_index("s")
    wid = sid * _NC + cid
    g0 = wid * tpw

    pltpu.sync_copy(len_hbm, len_v)
    pltpu.sync_copy(idx_hbm.at[pl.ds(g0 * 4, tpw * 4)], idx_all)
    lens_vec = len_v[...]
    lane = lax.iota(jnp.int32, _LANES)
    zeros = jnp.zeros((_LANES,), jnp.float32)

    for h in range(nslots // 2 // _LANES):
        qidx[pl.ds(h * _LANES, _LANES)] = (wid + _NW * (lane + h * _LANES)) >> 1
    pltpu.async_copy(idx2d_hbm.at[qidx], idx_mine, qsem)
    col0 = (wid % 2) * (8 * _T)

    for t in range(2 * _T):
        def zinit(dd, c, t=t):
            sl = pl.ds(pl.multiple_of(dd * _LANES, _LANES), _LANES)
            zbuf[t, sl] = zeros
            return c
        lax.fori_loop(0, d // _LANES, zinit, 0, unroll=4)

    lbs = [jnp.max(jnp.where(lane == s, lens_vec, 0)) for s in range(nsamp)]

    def lb_of(sq):
        r = lbs[0]
        for s in range(1, nsamp):
            r = jnp.where(sq == s, lbs[s], r)
        return r

    def slot_nv(j):
        gq = (wid + _NW * (j >> 1)) * (2 * _T) + (j & 1) * _T
        sq = gq // lsz
        return jnp.minimum(jnp.maximum(lb_of(sq) - (gq - sq * lsz), 0), _T)

    def pair_nv(p):
        gp = (wid + _NW * p) * (2 * _T)
        sq = gp // lsz
        return jnp.minimum(jnp.maximum(lb_of(sq) - (gp - sq * lsz), 0), 2 * _T)

    wpersamp = _NW // nsamp
    b = wid // wpersamp
    r0 = (wid % wpersamp) * tpw
    nv = jnp.minimum(jnp.maximum(lb_of(b) - r0, 0), tpw)

    def mask_grp(grp, carry):
        t = lane + grp * _LANES
        p = t * 4
        v = plsc.load_gather(idx_all, [p])
        for j in range(1, 4):
            v = v | plsc.load_gather(idx_all, [p + j])
        m = ((v != 0) & (t < nv)).astype(jnp.int32)
        mask_v[pl.ds(pl.multiple_of(grp * _LANES, _LANES), _LANES)] = m
        return carry

    lax.fori_loop(0, tpw // _LANES, mask_grp, 0)
    pltpu.sync_copy(mask_v, mask_hbm.at[pl.ds(g0, tpw)])

    pltpu.make_async_copy(idx2d_hbm.at[qidx], idx_mine, qsem).wait()

    def prep_gather(j, kr):
        nvq = slot_nv(j)
        rvec = jnp.full((_LANES,), j >> 1, jnp.int32)
        cb = col0 + (j & 1) * (4 * _T)
        for h in range(4 * _T // _LANES):
            tok = (lane >> 2) + 4 * h
            v = plsc.load_gather(idx_mine, [rvec, cb + lane + h * _LANES])
            gbufs[kr][pl.ds(h * _LANES, _LANES)] = jnp.where(tok < nvq, v, 0)

        @pl.when(nvq > 0)
        def _():
            pltpu.async_copy(table_hbm.at[gbufs[kr]], rows[kr], gsems[kr])

    def wait_gather(kr, nvq):
        @pl.when(nvq > 0)
        def _():
            pltpu.make_async_copy(table_hbm.at[gbufs[kr]], rows[kr],
                                  gsems[kr]).wait()

    def wait_write(p, ko):
        nvp = pair_nv(p)

        @pl.when(nvp > 0)
        def _():
            pltpu.make_async_copy(obufs[ko], out_hbm.at[pl.ds(0, 2 * _T), :],
                                  osems[ko]).wait()

        @pl.when(nvp == 0)
        def _():
            pltpu.make_async_copy(zbuf, out_hbm.at[pl.ds(0, 2 * _T), :],
                                  zsem).wait()

    def slot_step(s, k):
        kr = k % 2
        h = k & 1
        ko = (k >> 1) & 1
        nvq = slot_nv(s)
        p = s >> 1

        _when(s + 1 < nslots)(lambda: prep_gather(s + 1, 1 - kr))
        wait_gather(kr, nvq)
        if h == 0:
            _when(s >= 4)(lambda: wait_write(p - 2, ko))

        @pl.when(nvq > 0)
        def _():
            nsl = d // _LANES

            @plsc.parallel_loop(0, _T * nsl, unroll=4)
            def _(ii):
                t = ii // nsl
                dd = ii - t * nsl
                sl = pl.ds(pl.multiple_of(dd * _LANES, _LANES), _LANES)
                rws = rows[kr]
                obufs[ko][h * _T + t, sl] = (
                    (rws[4 * t, sl] + rws[4 * t + 1, sl]) +
                    (rws[4 * t + 2, sl] + rws[4 * t + 3, sl]))

        if h == 1:
            nvp = pair_nv(p)

            @pl.when((nvq == 0) & (nvp > 0))
            def _():
                for t in range(_T):
                    @plsc.parallel_loop(0, d // _LANES, unroll=4)
                    def _(dd, t=t):
                        sl = pl.ds(pl.multiple_of(dd * _LANES, _LANES),
                                   _LANES)
                        obufs[ko][_T + t, sl] = zeros

            gp = (wid + _NW * p) * (2 * _T)

            @pl.when(nvp > 0)
            def _():
                pltpu.async_copy(obufs[ko], out_hbm.at[pl.ds(gp, 2 * _T), :],
                                 osems[ko])

            @pl.when(nvp == 0)
            def _():
                pltpu.async_copy(zbuf, out_hbm.at[pl.ds(gp, 2 * _T), :], zsem)

    prep_gather(0, 0)
    ngroups = nslots // _GRP

    def group_body(i, carry):
        for k in range(_GRP):
            slot_step(_GRP * i + k, k)
        return carry

    lax.fori_loop(0, ngroups, group_body, 0)
    npairs = nslots // 2
    for p in (npairs - 2, npairs - 1):
        wait_write(p, p % 2)


@functools.partial(jax.jit, static_argnames=("nt", "d", "nsamp"))
def _compose_sc(idx_flat, idx2d, len_pad, table, *, nt, d, nsamp):
    tpw = nt // _NW
    nslots = tpw // _T
    mesh = plsc.VectorSubcoreMesh(
        core_axis_name="c", subcore_axis_name="s",
        num_cores=_NC, num_subcores=_NS)
    body = functools.partial(_compose_body, nt, d, tpw, nsamp)
    return pl.kernel(
        body,
        out_type=[
            jax.ShapeDtypeStruct((nt, d), jnp.float32),
            jax.ShapeDtypeStruct((nt,), jnp.int32),
        ],
        mesh=mesh,
        compiler_params=pltpu.CompilerParams(needs_layout_passes=False),
        scratch_types=[
            pltpu.VMEM((tpw * 4,), jnp.int32),
            pltpu.VMEM((nslots // 2, 16 * _T), jnp.int32),
            pltpu.VMEM((nslots // 2,), jnp.int32),
            pltpu.VMEM((tpw,), jnp.int32),
            pltpu.VMEM((_LANES,), jnp.int32),
            [pltpu.VMEM((4 * _T,), jnp.int32)] * 2,
            [pltpu.VMEM((4 * _T, d), jnp.float32)] * 2,
            [pltpu.VMEM((2 * _T, d), jnp.float32)] * 2,
            pltpu.VMEM((2 * _T, d), jnp.float32),
            pltpu.SemaphoreType.DMA,
            [pltpu.SemaphoreType.DMA] * 2,
            [pltpu.SemaphoreType.DMA] * 2,
            pltpu.SemaphoreType.DMA,
        ],
    )(idx_flat, idx2d, len_pad, table)


def kernel(phoneme_flat, phoneme_token_len, table):
    bsz, pt = phoneme_flat.shape
    lx = pt // 4
    nt = bsz * lx
    d = table.shape[1]
    idx_flat = phoneme_flat.reshape(-1).astype(jnp.int32)
    idx2d = idx_flat.reshape(nt // (4 * _T), 16 * _T)
    len_pad = jnp.zeros((_LANES,), jnp.int32).at[:bsz].set(
        phoneme_token_len.astype(jnp.int32))
    out_flat, mask_i = _compose_sc(idx_flat, idx2d, len_pad, table,
                                   nt=nt, d=d, nsamp=bsz)
    out = out_flat.reshape(bsz, lx, d)
    pf_mask = mask_i.reshape(bsz, lx).astype(bool)
    return out, pf_mask
